# baseline (speedup 1.0000x reference)
import jax
import jax.numpy as jnp
from jax.experimental import pallas as pl
from jax.experimental.pallas import tpu as pltpu

_SUBLANE = 8
_LANE = 128
_GROUP = 16
_PACK = _LANE // _GROUP
_TILE_THRESHOLD = 1024


def _round_up(n, m):
    return ((n + m - 1) // m) * m


def _body_packed(has_prev):
    def compute(x_ref, w_ref, o_ref):
        x = x_ref[...]
        for u in range(_PACK):
            o_ref[u] = jax.lax.dot_general(
                x,
                w_ref[u * _LANE:(u + 1) * _LANE, :],
                dimension_numbers=(((1,), (0,)), ((), ())),
                preferred_element_type=jnp.float32,
            ).astype(o_ref.dtype)

    if not has_prev:
        return compute

    def body_prev(x_ref, w_ref, prev_ref, o_ref):
        del prev_ref
        compute(x_ref, w_ref, o_ref)

    return body_prev


def _forward_packed_chunks(xds, w8, tile_d, d):
    dc = d // len(xds)
    out3 = None
    for c, xdc in enumerate(xds):
        off = c * (dc // tile_d)
        in_specs = [
            pl.BlockSpec((tile_d, _LANE), lambda i: (i, 0)),
            pl.BlockSpec((_PACK * _LANE, _LANE), lambda i: (0, 0)),
        ]
        operands = [xdc, w8]
        aliases = {}
        if out3 is not None:
            in_specs.append(
                pl.BlockSpec((_PACK, _SUBLANE, _LANE), lambda i: (0, 0, 0)))
            operands.append(out3)
            aliases = {2: 0}
        out3 = pl.pallas_call(
            _body_packed(out3 is not None),
            out_shape=jax.ShapeDtypeStruct((_PACK, d, _LANE), xdc.dtype),
            grid=(dc // tile_d,),
            in_specs=in_specs,
            out_specs=pl.BlockSpec((_PACK, tile_d, _LANE),
                                   lambda i, _o=off: (0, _o + i, 0)),
            input_output_aliases=aliases,
            compiler_params=pltpu.CompilerParams(
                dimension_semantics=("arbitrary",)
            ),
            cost_estimate=pl.CostEstimate(
                flops=2 * dc * _PACK * _LANE * _LANE,
                transcendentals=0,
                bytes_accessed=4 * (dc * _LANE + _PACK * dc * _LANE),
            ),
        )(*operands)
    return out3.reshape(_PACK * d, _LANE)


def _prepare_w8(wt_pad):
    w16 = wt_pad[0:_GROUP, :]
    blocks = [
        jnp.pad(w16, ((_GROUP * u, _LANE - _GROUP * (u + 1)), (0, 0)))
        for u in range(_PACK)
    ]
    return jnp.concatenate(blocks, axis=0)


def _body2d(in_dim):
    def body(x_ref, w_ref, o_ref):
        o_ref[...] = jax.lax.dot_general(
            x_ref[...],
            w_ref[0:in_dim, :],
            dimension_numbers=(((1,), (0,)), ((), ())),
            preferred_element_type=jnp.float32,
        ).astype(o_ref.dtype)

    return body


def _forward2d(x, wt_pad, rows_pad, tile_rows):
    rows, in_dim = x.shape
    in_pad, out_pad = wt_pad.shape
    if rows_pad != rows:
        x = jnp.pad(x, ((0, rows_pad - rows), (0, 0)))
    grid = (rows_pad // tile_rows,)
    return pl.pallas_call(
        _body2d(in_dim),
        out_shape=jax.ShapeDtypeStruct((rows_pad, out_pad), x.dtype),
        grid=grid,
        in_specs=[
            pl.BlockSpec((tile_rows, in_dim), lambda i: (i, 0)),
            pl.BlockSpec((in_pad, out_pad), lambda i: (0, 0)),
        ],
        out_specs=pl.BlockSpec((tile_rows, out_pad), lambda i: (i, 0)),
        compiler_params=pltpu.CompilerParams(
            dimension_semantics=("parallel",)
        ),
    )(x, wt_pad)


@jax.jit
def kernel(xs, wt_pad):
    n, b, in_dim = xs.shape
    rows = n * b
    if rows < _TILE_THRESHOLD:
        rows_pad = _round_up(max(rows, _SUBLANE), _SUBLANE)
        return _forward2d(xs.reshape(rows, in_dim), wt_pad, rows_pad,
                          rows_pad)
    if rows % 512 == 0 and in_dim <= _GROUP:
        d = rows // _PACK
        tile_d = 64
        for cand in (2048, 1024, 512, 256, 128):
            if d % cand == 0:
                tile_d = cand
                break
        n_chunks = 1
        for cand in (4, 2):
            if d % (cand * tile_d) == 0:
                n_chunks = cand
                break
        dc = d // n_chunks
        x3 = xs.reshape(_PACK, d, in_dim)
        xds = []
        for c in range(n_chunks):
            x3c = x3[:, c * dc:(c + 1) * dc, :]
            xpc = jnp.pad(x3c, ((0, 0), (0, 0), (0, _GROUP - in_dim)))
            xds.append(
                jax.lax.reshape(xpc, (dc, _LANE), dimensions=(1, 0, 2)))
        w8 = _prepare_w8(wt_pad)
        return _forward_packed_chunks(xds, w8, tile_d, d)
    rows_pad = _round_up(rows, 512)
    return _forward2d(xs.reshape(rows, in_dim), wt_pad, rows_pad, 512)


# speedup vs baseline: 1.0783x; 1.0783x over previous
import jax
import jax.numpy as jnp
from jax.experimental import pallas as pl
from jax.experimental.pallas import tpu as pltpu

_SUBLANE = 8
_LANE = 128
_GROUP = 16
_PACK = _LANE // _GROUP
_TILE_THRESHOLD = 1024


def _round_up(n, m):
    return ((n + m - 1) // m) * m


def _body_packed(has_prev):
    def compute(x_ref, w_ref, o_ref):
        x = x_ref[...]
        for u in range(_PACK):
            o_ref[u] = jax.lax.dot_general(
                x,
                w_ref[u * _LANE:(u + 1) * _LANE, :],
                dimension_numbers=(((1,), (0,)), ((), ())),
                preferred_element_type=jnp.float32,
            ).astype(o_ref.dtype)

    if not has_prev:
        return compute

    def body_prev(x_ref, w_ref, prev_ref, o_ref):
        del prev_ref
        compute(x_ref, w_ref, o_ref)

    return body_prev


def _forward_packed_chunks(xds, w8, tile_d, d):
    dc = d // len(xds)
    out3 = None
    for c, xdc in enumerate(xds):
        off = c * (dc // tile_d)
        in_specs = [
            pl.BlockSpec((tile_d, _LANE), lambda i: (i, 0)),
            pl.BlockSpec((_PACK * _LANE, _LANE), lambda i: (0, 0)),
        ]
        operands = [xdc, w8]
        aliases = {}
        if out3 is not None:
            in_specs.append(
                pl.BlockSpec((_PACK, _SUBLANE, _LANE), lambda i: (0, 0, 0)))
            operands.append(out3)
            aliases = {2: 0}
        out3 = pl.pallas_call(
            _body_packed(out3 is not None),
            out_shape=jax.ShapeDtypeStruct((_PACK, d, _LANE), xdc.dtype),
            grid=(dc // tile_d,),
            in_specs=in_specs,
            out_specs=pl.BlockSpec((_PACK, tile_d, _LANE),
                                   lambda i, _o=off: (0, _o + i, 0)),
            input_output_aliases=aliases,
            compiler_params=pltpu.CompilerParams(
                dimension_semantics=("arbitrary",)
            ),
            cost_estimate=pl.CostEstimate(
                flops=2 * dc * _PACK * _LANE * _LANE,
                transcendentals=0,
                bytes_accessed=4 * (dc * _LANE + _PACK * dc * _LANE),
            ),
        )(*operands)
    return out3.reshape(_PACK * d, _LANE)


def _prepare_w8(wt_pad):
    w16 = wt_pad[0:_GROUP, :]
    blocks = [
        jnp.pad(w16, ((_GROUP * u, _LANE - _GROUP * (u + 1)), (0, 0)))
        for u in range(_PACK)
    ]
    return jnp.concatenate(blocks, axis=0)


def _body2d(in_dim):
    def body(x_ref, w_ref, o_ref):
        o_ref[...] = jax.lax.dot_general(
            x_ref[...],
            w_ref[0:in_dim, :],
            dimension_numbers=(((1,), (0,)), ((), ())),
            preferred_element_type=jnp.float32,
        ).astype(o_ref.dtype)

    return body


def _forward2d(x, wt_pad, rows_pad, tile_rows):
    rows, in_dim = x.shape
    in_pad, out_pad = wt_pad.shape
    if rows_pad != rows:
        x = jnp.pad(x, ((0, rows_pad - rows), (0, 0)))
    grid = (rows_pad // tile_rows,)
    return pl.pallas_call(
        _body2d(in_dim),
        out_shape=jax.ShapeDtypeStruct((rows_pad, out_pad), x.dtype),
        grid=grid,
        in_specs=[
            pl.BlockSpec((tile_rows, in_dim), lambda i: (i, 0)),
            pl.BlockSpec((in_pad, out_pad), lambda i: (0, 0)),
        ],
        out_specs=pl.BlockSpec((tile_rows, out_pad), lambda i: (i, 0)),
        compiler_params=pltpu.CompilerParams(
            dimension_semantics=("parallel",)
        ),
    )(x, wt_pad)


@jax.jit
def kernel(xs, wt_pad):
    n, b, in_dim = xs.shape
    rows = n * b
    if rows < _TILE_THRESHOLD:
        rows_pad = _round_up(max(rows, _SUBLANE), _SUBLANE)
        return _forward2d(xs.reshape(rows, in_dim), wt_pad, rows_pad,
                          rows_pad)
    if rows % 512 == 0 and in_dim <= _GROUP:
        d = rows // _PACK
        tile_d = 64
        for cand in (2048, 1024, 512, 256, 128):
            if d % cand == 0:
                tile_d = cand
                break
        x3 = xs.reshape(_PACK, d, in_dim)
        xp = jnp.pad(x3, ((0, 0), (0, 0), (0, _GROUP - in_dim)))
        xd = jax.lax.reshape(xp, (d, _LANE), dimensions=(1, 0, 2))
        w8 = _prepare_w8(wt_pad)
        return _forward_packed_chunks([xd], w8, tile_d, d)
    rows_pad = _round_up(rows, 512)
    return _forward2d(xs.reshape(rows, in_dim), wt_pad, rows_pad, 512)


# speedup vs baseline: 1.2317x; 1.1423x over previous
import jax
import jax.numpy as jnp
from jax.experimental import pallas as pl
from jax.experimental.pallas import tpu as pltpu

_SUBLANE = 8
_LANE = 128
_GROUP = 16
_PACK = _LANE // _GROUP
_TILE_THRESHOLD = 1024


def _round_up(n, m):
    return ((n + m - 1) // m) * m


def _body_packed(has_prev):
    def compute(x_ref, w_ref, o_ref):
        x = x_ref[...]
        for u in range(_PACK):
            o_ref[u] = jax.lax.dot_general(
                x,
                w_ref[u * _LANE:(u + 1) * _LANE, :],
                dimension_numbers=(((1,), (0,)), ((), ())),
                preferred_element_type=jnp.float32,
            ).astype(o_ref.dtype)

    if not has_prev:
        return compute

    def body_prev(x_ref, w_ref, prev_ref, o_ref):
        del prev_ref
        compute(x_ref, w_ref, o_ref)

    return body_prev


def _forward_packed_chunks(xds, w8, tile_d, d, out_dtype=jnp.float32):
    dc = d // len(xds)
    out3 = None
    for c, xdc in enumerate(xds):
        off = c * (dc // tile_d)
        in_specs = [
            pl.BlockSpec((tile_d, _LANE), lambda i: (i, 0)),
            pl.BlockSpec((_PACK * _LANE, _LANE), lambda i: (0, 0)),
        ]
        operands = [xdc, w8]
        aliases = {}
        if out3 is not None:
            in_specs.append(
                pl.BlockSpec((_PACK, _SUBLANE, _LANE), lambda i: (0, 0, 0)))
            operands.append(out3)
            aliases = {2: 0}
        out3 = pl.pallas_call(
            _body_packed(out3 is not None),
            out_shape=jax.ShapeDtypeStruct((_PACK, d, _LANE), out_dtype),
            grid=(dc // tile_d,),
            in_specs=in_specs,
            out_specs=pl.BlockSpec((_PACK, tile_d, _LANE),
                                   lambda i, _o=off: (0, _o + i, 0)),
            input_output_aliases=aliases,
            compiler_params=pltpu.CompilerParams(
                dimension_semantics=("arbitrary",)
            ),
            cost_estimate=pl.CostEstimate(
                flops=2 * dc * _PACK * _LANE * _LANE,
                transcendentals=0,
                bytes_accessed=4 * (dc * _LANE + _PACK * dc * _LANE),
            ),
        )(*operands)
    return out3.reshape(_PACK * d, _LANE)


def _prepare_w8(wt_pad):
    w16 = wt_pad[0:_GROUP, :]
    blocks = [
        jnp.pad(w16, ((_GROUP * u, _LANE - _GROUP * (u + 1)), (0, 0)))
        for u in range(_PACK)
    ]
    return jnp.concatenate(blocks, axis=0)


def _body2d(in_dim):
    def body(x_ref, w_ref, o_ref):
        o_ref[...] = jax.lax.dot_general(
            x_ref[...],
            w_ref[0:in_dim, :],
            dimension_numbers=(((1,), (0,)), ((), ())),
            preferred_element_type=jnp.float32,
        ).astype(o_ref.dtype)

    return body


def _forward2d(x, wt_pad, rows_pad, tile_rows):
    rows, in_dim = x.shape
    in_pad, out_pad = wt_pad.shape
    if rows_pad != rows:
        x = jnp.pad(x, ((0, rows_pad - rows), (0, 0)))
    grid = (rows_pad // tile_rows,)
    return pl.pallas_call(
        _body2d(in_dim),
        out_shape=jax.ShapeDtypeStruct((rows_pad, out_pad), x.dtype),
        grid=grid,
        in_specs=[
            pl.BlockSpec((tile_rows, in_dim), lambda i: (i, 0)),
            pl.BlockSpec((in_pad, out_pad), lambda i: (0, 0)),
        ],
        out_specs=pl.BlockSpec((tile_rows, out_pad), lambda i: (i, 0)),
        compiler_params=pltpu.CompilerParams(
            dimension_semantics=("parallel",)
        ),
    )(x, wt_pad)


@jax.jit
def kernel(xs, wt_pad):
    n, b, in_dim = xs.shape
    rows = n * b
    if rows < _TILE_THRESHOLD:
        rows_pad = _round_up(max(rows, _SUBLANE), _SUBLANE)
        return _forward2d(xs.reshape(rows, in_dim), wt_pad, rows_pad,
                          rows_pad)
    if rows % 512 == 0 and in_dim <= _GROUP:
        d = rows // _PACK
        tile_d = 64
        for cand in (2048, 1024, 512, 256, 128):
            if d % cand == 0:
                tile_d = cand
                break
        x3 = xs.reshape(_PACK, d, in_dim)
        xp = jnp.pad(x3, ((0, 0), (0, 0), (0, _GROUP - in_dim)))
        xpb = xp.astype(jnp.bfloat16)
        xd = jax.lax.reshape(xpb, (d, _LANE), dimensions=(1, 0, 2))
        w8 = _prepare_w8(wt_pad).astype(jnp.bfloat16)
        return _forward_packed_chunks([xd], w8, tile_d, d)
    rows_pad = _round_up(rows, 512)
    return _forward2d(xs.reshape(rows, in_dim), wt_pad, rows_pad, 512)


# speedup vs baseline: 1.2430x; 1.0092x over previous
import jax
import jax.numpy as jnp
from jax.experimental import pallas as pl
from jax.experimental.pallas import tpu as pltpu

_SUBLANE = 8
_LANE = 128
_GROUP = 16
_PACK = _LANE // _GROUP
_TILE_THRESHOLD = 1024


def _round_up(n, m):
    return ((n + m - 1) // m) * m


def _body_packed(has_prev):
    def compute(x_ref, w_ref, o_ref):
        x = x_ref[...]
        for u in range(_PACK):
            o_ref[u] = jax.lax.dot_general(
                x,
                w_ref[u * _LANE:(u + 1) * _LANE, :],
                dimension_numbers=(((1,), (0,)), ((), ())),
                preferred_element_type=jnp.float32,
            ).astype(o_ref.dtype)

    if not has_prev:
        return compute

    def body_prev(x_ref, w_ref, prev_ref, o_ref):
        del prev_ref
        compute(x_ref, w_ref, o_ref)

    return body_prev


def _forward_packed_chunks(xds, w8, tile_d, d, out_dtype=jnp.float32):
    dc = d // len(xds)
    out3 = None
    for c, xdc in enumerate(xds):
        off = c * (dc // tile_d)
        in_specs = [
            pl.BlockSpec((tile_d, _LANE), lambda i: (i, 0)),
            pl.BlockSpec((_PACK * _LANE, _LANE), lambda i: (0, 0)),
        ]
        operands = [xdc, w8]
        aliases = {}
        if out3 is not None:
            in_specs.append(
                pl.BlockSpec((_PACK, _SUBLANE, _LANE), lambda i: (0, 0, 0)))
            operands.append(out3)
            aliases = {2: 0}
        out3 = pl.pallas_call(
            _body_packed(out3 is not None),
            out_shape=jax.ShapeDtypeStruct((_PACK, d, _LANE), out_dtype),
            grid=(dc // tile_d,),
            in_specs=in_specs,
            out_specs=pl.BlockSpec((_PACK, tile_d, _LANE),
                                   lambda i, _o=off: (0, _o + i, 0)),
            input_output_aliases=aliases,
            compiler_params=pltpu.CompilerParams(
                dimension_semantics=("arbitrary",)
            ),
            cost_estimate=pl.CostEstimate(
                flops=2 * dc * _PACK * _LANE * _LANE,
                transcendentals=0,
                bytes_accessed=4 * (dc * _LANE + _PACK * dc * _LANE),
            ),
        )(*operands)
    return out3.reshape(_PACK * d, _LANE)


def _prepare_w8(wt_pad):
    w16 = wt_pad[0:_GROUP, :]
    blocks = [
        jnp.pad(w16, ((_GROUP * u, _LANE - _GROUP * (u + 1)), (0, 0)))
        for u in range(_PACK)
    ]
    return jnp.concatenate(blocks, axis=0)


def _body2d(in_dim):
    def body(x_ref, w_ref, o_ref):
        o_ref[...] = jax.lax.dot_general(
            x_ref[...],
            w_ref[0:in_dim, :],
            dimension_numbers=(((1,), (0,)), ((), ())),
            preferred_element_type=jnp.float32,
        ).astype(o_ref.dtype)

    return body


def _forward2d(x, wt_pad, rows_pad, tile_rows):
    rows, in_dim = x.shape
    in_pad, out_pad = wt_pad.shape
    if rows_pad != rows:
        x = jnp.pad(x, ((0, rows_pad - rows), (0, 0)))
    grid = (rows_pad // tile_rows,)
    return pl.pallas_call(
        _body2d(in_dim),
        out_shape=jax.ShapeDtypeStruct((rows_pad, out_pad), x.dtype),
        grid=grid,
        in_specs=[
            pl.BlockSpec((tile_rows, in_dim), lambda i: (i, 0)),
            pl.BlockSpec((in_pad, out_pad), lambda i: (0, 0)),
        ],
        out_specs=pl.BlockSpec((tile_rows, out_pad), lambda i: (i, 0)),
        compiler_params=pltpu.CompilerParams(
            dimension_semantics=("parallel",)
        ),
    )(x, wt_pad)


@jax.jit
def kernel(xs, wt_pad):
    n, b, in_dim = xs.shape
    rows = n * b
    if rows < _TILE_THRESHOLD:
        rows_pad = _round_up(max(rows, _SUBLANE), _SUBLANE)
        return _forward2d(xs.reshape(rows, in_dim), wt_pad, rows_pad,
                          rows_pad)
    if rows % 512 == 0 and in_dim <= _GROUP:
        d = rows // _PACK
        tile_d = 64
        for cand in (4096, 2048, 1024, 512, 256, 128):
            if d % cand == 0:
                tile_d = cand
                break
        x3 = xs.reshape(_PACK, d, in_dim)
        xp = jnp.pad(x3, ((0, 0), (0, 0), (0, _GROUP - in_dim)))
        xpb = xp.astype(jnp.bfloat16)
        xd = jax.lax.reshape(xpb, (d, _LANE), dimensions=(1, 0, 2))
        w8 = _prepare_w8(wt_pad).astype(jnp.bfloat16)
        return _forward_packed_chunks([xd], w8, tile_d, d)
    rows_pad = _round_up(rows, 512)
    return _forward2d(xs.reshape(rows, in_dim), wt_pad, rows_pad, 512)
